# baseline (speedup 1.0000x reference)
import functools

import jax
import jax.numpy as jnp
from jax import lax
from jax.experimental import pallas as pl
from jax.experimental.pallas import tpu as pltpu
from jax.experimental.pallas import tpu_sc as plsc

EDIM = 64
NFIELDS = 7
D_MODEL = 512

_NC = 2
_NS = 16
_NW = _NC * _NS
_CHUNK = 128
_TPW = 256
_BK = 32768
_HB = _BK // 2


def _tr_body(t_ref, o_ref):
    x = t_ref[...]
    o_ref[...] = jnp.concatenate([x[:, :_HB].T, x[:, _HB:].T], axis=1)


@functools.cache
def _make_transpose(vocab):
    grid = (vocab + _BK - 1) // _BK
    return pl.pallas_call(
        _tr_body,
        grid=(grid,),
        in_specs=[pl.BlockSpec((EDIM, _BK), lambda g: (0, g))],
        out_specs=pl.BlockSpec((_HB, 2 * EDIM), lambda g: (g, 0)),
        out_shape=jax.ShapeDtypeStruct((grid * _HB, 2 * EDIM), jnp.float32),
    )


def _gather_body(xti, t2, out, idx_v, rows_v, gsem):
    wid = lax.axis_index("s") * _NC + lax.axis_index("c")
    base = wid * _TPW
    pltpu.sync_copy(xti.at[wid], idx_v)
    gs = [
        pltpu.async_copy(
            t2.at[idx_v.at[c]],
            rows_v.at[pl.ds(c * _CHUNK, _CHUNK)],
            gsem,
        )
        for c in range(_TPW // _CHUNK)
    ]
    for g in gs:
        g.wait()
    pltpu.sync_copy(rows_v, out.at[pl.ds(base, _TPW)])


@functools.cache
def _make_gather(n_tokens, n_lines):
    mesh = plsc.VectorSubcoreMesh(core_axis_name="c", subcore_axis_name="s")
    return functools.partial(
        pl.kernel,
        out_type=jax.ShapeDtypeStruct((n_tokens, 2 * EDIM), jnp.float32),
        mesh=mesh,
        scratch_types=[
            pltpu.VMEM((8, _CHUNK), jnp.int32),
            pltpu.VMEM((_TPW, 2 * EDIM), jnp.float32),
            pltpu.SemaphoreType.DMA,
        ],
        compiler_params=pltpu.CompilerParams(use_tc_tiling_on_sc=True),
    )(_gather_body)


def _mm_body(h0, h1, h2, h3, h4, h5, h6, m_ref, w_ref, b_ref, o_ref):
    acc = b_ref[...].astype(jnp.float32)
    tm = o_ref.shape[0]
    for i, h_ref in enumerate((h0, h1, h2, h3, h4, h5, h6)):
        wide = h_ref[...]
        sel = m_ref[i].reshape(tm, 1) > 0.5
        h_i = jnp.where(sel, wide[:, EDIM:], wide[:, :EDIM]).astype(
            jnp.bfloat16)
        acc = acc + jnp.dot(h_i, w_ref[i], preferred_element_type=jnp.float32)
    o_ref[...] = acc


@functools.cache
def _make_matmul(n_tokens, tm):
    h_spec = pl.BlockSpec((tm, 2 * EDIM), lambda m: (m, 0))
    return pl.pallas_call(
        _mm_body,
        grid=(n_tokens // tm,),
        in_specs=[h_spec] * NFIELDS + [
            pl.BlockSpec((NFIELDS, tm), lambda m: (0, m)),
            pl.BlockSpec((NFIELDS, EDIM, D_MODEL), lambda m: (0, 0, 0)),
            pl.BlockSpec((1, D_MODEL), lambda m: (0, 0)),
        ],
        out_specs=pl.BlockSpec((tm, D_MODEL), lambda m: (m, 0)),
        out_shape=jax.ShapeDtypeStruct((n_tokens, D_MODEL), jnp.float32),
    )


def kernel(x, T0, T1, T2, T3, T4, T5, T6, W, b):
    bsz, seq, nf = x.shape
    n = bsz * seq
    xr = x.reshape(n, NFIELDS).astype(jnp.int32)
    line = (xr // _BK) * _HB + (xr & (_HB - 1))
    half = ((xr // _HB) & 1).astype(jnp.float32)
    xt = jnp.pad(
        line.T.reshape(NFIELDS, _NW, _TPW // _CHUNK, _CHUNK),
        ((0, 0), (0, 0), (0, 8 - _TPW // _CHUNK), (0, 0)),
    )
    m = half.T
    hs = []
    for i, T in enumerate((T0, T1, T2, T3, T4, T5, T6)):
        t2 = _make_transpose(T.shape[0])(jnp.transpose(T))
        hs.append(_make_gather(n, t2.shape[0])(xt[i], t2))
    wt = W.T.reshape(NFIELDS, EDIM, D_MODEL).astype(jnp.bfloat16)
    out = _make_matmul(n, 512)(*hs, m, wt, b.reshape(1, D_MODEL))
    return out.reshape(bsz, seq, D_MODEL)


# speedup vs baseline: 1.1698x; 1.1698x over previous
import functools

import jax
import jax.numpy as jnp
from jax import lax
from jax.experimental import pallas as pl
from jax.experimental.pallas import tpu as pltpu
from jax.experimental.pallas import tpu_sc as plsc

EDIM = 64
NFIELDS = 7
D_MODEL = 512

_NC = 2
_NS = 16
_NW = _NC * _NS
_CHUNK = 128
_TPW = 256
_BK = 14336
_HB = _BK // 2


def _tr_body(t_ref, o_ref):
    x = t_ref[...]
    o_ref[...] = jnp.concatenate([x[:, :_HB].T, x[:, _HB:].T], axis=1)


@functools.cache
def _make_transpose(vocab):
    grid = (vocab + _BK - 1) // _BK
    return pl.pallas_call(
        _tr_body,
        grid=(grid,),
        in_specs=[pl.BlockSpec((EDIM, _BK), lambda g: (0, g))],
        out_specs=pl.BlockSpec((_HB, 2 * EDIM), lambda g: (g, 0)),
        out_shape=jax.ShapeDtypeStruct((grid * _HB, 2 * EDIM), jnp.float32),
    )


def _gather_body(xti, t2, out, idx_v, rows_v, gsem):
    wid = lax.axis_index("s") * _NC + lax.axis_index("c")
    base = wid * _TPW
    pltpu.sync_copy(xti.at[wid], idx_v)
    gs = [
        pltpu.async_copy(
            t2.at[idx_v.at[c]],
            rows_v.at[pl.ds(c * _CHUNK, _CHUNK)],
            gsem,
        )
        for c in range(_TPW // _CHUNK)
    ]
    for g in gs:
        g.wait()
    pltpu.sync_copy(rows_v, out.at[pl.ds(base, _TPW)])


@functools.cache
def _make_gather(n_tokens, n_lines):
    mesh = plsc.VectorSubcoreMesh(core_axis_name="c", subcore_axis_name="s")
    return functools.partial(
        pl.kernel,
        out_type=jax.ShapeDtypeStruct((n_tokens, 2 * EDIM), jnp.float32),
        mesh=mesh,
        scratch_types=[
            pltpu.VMEM((8, _CHUNK), jnp.int32),
            pltpu.VMEM((_TPW, 2 * EDIM), jnp.float32),
            pltpu.SemaphoreType.DMA,
        ],
        compiler_params=pltpu.CompilerParams(use_tc_tiling_on_sc=True),
    )(_gather_body)


def _mm_body(h0, h1, h2, h3, h4, h5, h6, m_ref, w_ref, b_ref, o_ref):
    acc = b_ref[...].astype(jnp.float32)
    tm = o_ref.shape[0]
    for i, h_ref in enumerate((h0, h1, h2, h3, h4, h5, h6)):
        wide = h_ref[...]
        sel = m_ref[i].reshape(tm, 1) > 0.5
        h_i = jnp.where(sel, wide[:, EDIM:], wide[:, :EDIM]).astype(
            jnp.bfloat16)
        acc = acc + jnp.dot(h_i, w_ref[i], preferred_element_type=jnp.float32)
    o_ref[...] = acc


@functools.cache
def _make_matmul(n_tokens, tm):
    h_spec = pl.BlockSpec((tm, 2 * EDIM), lambda m: (m, 0))
    return pl.pallas_call(
        _mm_body,
        grid=(n_tokens // tm,),
        in_specs=[h_spec] * NFIELDS + [
            pl.BlockSpec((NFIELDS, tm), lambda m: (0, m)),
            pl.BlockSpec((NFIELDS, EDIM, D_MODEL), lambda m: (0, 0, 0)),
            pl.BlockSpec((1, D_MODEL), lambda m: (0, 0)),
        ],
        out_specs=pl.BlockSpec((tm, D_MODEL), lambda m: (m, 0)),
        out_shape=jax.ShapeDtypeStruct((n_tokens, D_MODEL), jnp.float32),
    )


def kernel(x, T0, T1, T2, T3, T4, T5, T6, W, b):
    bsz, seq, nf = x.shape
    n = bsz * seq
    xr = x.reshape(n, NFIELDS).astype(jnp.int32)
    line = (xr // _BK) * _HB + (xr & (_HB - 1))
    half = ((xr // _HB) & 1).astype(jnp.float32)
    xt = jnp.pad(
        line.T.reshape(NFIELDS, _NW, _TPW // _CHUNK, _CHUNK),
        ((0, 0), (0, 0), (0, 8 - _TPW // _CHUNK), (0, 0)),
    )
    m = half.T
    hs = []
    for i, T in enumerate((T0, T1, T2, T3, T4, T5, T6)):
        t2 = _make_transpose(T.shape[0])(jnp.transpose(T))
        hs.append(_make_gather(n, t2.shape[0])(xt[i], t2))
    wt = W.T.reshape(NFIELDS, EDIM, D_MODEL).astype(jnp.bfloat16)
    out = _make_matmul(n, 512)(*hs, m, wt, b.reshape(1, D_MODEL))
    return out.reshape(bsz, seq, D_MODEL)


# speedup vs baseline: 1.1727x; 1.0025x over previous
import functools

import jax
import jax.numpy as jnp
from jax import lax
from jax.experimental import pallas as pl
from jax.experimental.pallas import tpu as pltpu
from jax.experimental.pallas import tpu_sc as plsc

EDIM = 64
NFIELDS = 7
D_MODEL = 512

_NC = 2
_NS = 16
_NW = _NC * _NS
_CHUNK = 128
_TPW = 256
_BK = 14336
_HB = _BK // 2


def _tr_body(t_ref, o_ref):
    x = t_ref[...]
    o_ref[...] = jnp.concatenate([x[:, :_HB].T, x[:, _HB:].T], axis=1)


@functools.cache
def _make_transpose(vocab):
    grid = (vocab + _BK - 1) // _BK
    return pl.pallas_call(
        _tr_body,
        grid=(grid,),
        in_specs=[pl.BlockSpec((EDIM, _BK), lambda g: (0, g))],
        out_specs=pl.BlockSpec((_HB, 2 * EDIM), lambda g: (g, 0)),
        out_shape=jax.ShapeDtypeStruct((grid * _HB, 2 * EDIM), jnp.float32),
    )


def _gather_body(xti, t2, out, idx_v, rows_v, gsem):
    wid = lax.axis_index("s") * _NC + lax.axis_index("c")
    base = wid * _TPW
    pltpu.sync_copy(xti.at[wid], idx_v)
    gs = [
        pltpu.async_copy(
            t2.at[idx_v.at[c]],
            rows_v.at[pl.ds(c * _CHUNK, _CHUNK)],
            gsem,
        )
        for c in range(_TPW // _CHUNK)
    ]
    for g in gs:
        g.wait()
    pltpu.sync_copy(rows_v, out.at[pl.ds(base, _TPW)])


@functools.cache
def _make_gather(n_tokens, n_lines):
    mesh = plsc.VectorSubcoreMesh(core_axis_name="c", subcore_axis_name="s")
    return functools.partial(
        pl.kernel,
        out_type=jax.ShapeDtypeStruct((n_tokens, 2 * EDIM), jnp.float32),
        mesh=mesh,
        scratch_types=[
            pltpu.VMEM((8, _CHUNK), jnp.int32),
            pltpu.VMEM((_TPW, 2 * EDIM), jnp.float32),
            pltpu.SemaphoreType.DMA,
        ],
        compiler_params=pltpu.CompilerParams(use_tc_tiling_on_sc=True),
    )(_gather_body)


def _mm_body(h0, h1, h2, h3, h4, h5, h6, m_ref, w_ref, b_ref, o_ref):
    acc = b_ref[...].astype(jnp.float32)
    tm = o_ref.shape[0]
    for i, h_ref in enumerate((h0, h1, h2, h3, h4, h5, h6)):
        wide = h_ref[...]
        sel = m_ref[i].reshape(tm, 1) > 0.5
        h_i = jnp.where(sel, wide[:, EDIM:], wide[:, :EDIM]).astype(
            jnp.bfloat16)
        acc = acc + jnp.dot(h_i, w_ref[i], preferred_element_type=jnp.float32)
    o_ref[...] = acc


@functools.cache
def _make_matmul(n_tokens, tm):
    h_spec = pl.BlockSpec((tm, 2 * EDIM), lambda m: (m, 0))
    return pl.pallas_call(
        _mm_body,
        grid=(n_tokens // tm,),
        in_specs=[h_spec] * NFIELDS + [
            pl.BlockSpec((NFIELDS, tm), lambda m: (0, m)),
            pl.BlockSpec((NFIELDS, EDIM, D_MODEL), lambda m: (0, 0, 0)),
            pl.BlockSpec((1, D_MODEL), lambda m: (0, 0)),
        ],
        out_specs=pl.BlockSpec((tm, D_MODEL), lambda m: (m, 0)),
        out_shape=jax.ShapeDtypeStruct((n_tokens, D_MODEL), jnp.float32),
    )


def kernel(x, T0, T1, T2, T3, T4, T5, T6, W, b):
    bsz, seq, nf = x.shape
    n = bsz * seq
    xr = x.reshape(n, NFIELDS).astype(jnp.int32)
    line = (xr // _BK) * _HB + (xr % _HB)
    half = ((xr // _HB) & 1).astype(jnp.float32)
    xt = jnp.pad(
        line.T.reshape(NFIELDS, _NW, _TPW // _CHUNK, _CHUNK),
        ((0, 0), (0, 0), (0, 8 - _TPW // _CHUNK), (0, 0)),
    )
    m = half.T
    hs = []
    for i, T in enumerate((T0, T1, T2, T3, T4, T5, T6)):
        t2 = _make_transpose(T.shape[0])(jnp.transpose(T))
        hs.append(_make_gather(n, t2.shape[0])(xt[i], t2))
    wt = W.T.reshape(NFIELDS, EDIM, D_MODEL).astype(jnp.bfloat16)
    out = _make_matmul(n, 512)(*hs, m, wt, b.reshape(1, D_MODEL))
    return out.reshape(bsz, seq, D_MODEL)


# speedup vs baseline: 1.1951x; 1.0191x over previous
import functools

import jax
import jax.numpy as jnp
from jax import lax
from jax.experimental import pallas as pl
from jax.experimental.pallas import tpu as pltpu
from jax.experimental.pallas import tpu_sc as plsc

EDIM = 64
NFIELDS = 7
D_MODEL = 512

_NC = 2
_NS = 16
_NW = _NC * _NS
_CHUNK = 128
_TPW = 256
_BK = 25088
_HB = _BK // 2


def _tr_body(t_ref, o_ref):
    x = t_ref[...]
    o_ref[...] = jnp.concatenate([x[:, :_HB].T, x[:, _HB:].T], axis=1)


@functools.cache
def _make_transpose(vocab):
    grid = (vocab + _BK - 1) // _BK
    return pl.pallas_call(
        _tr_body,
        grid=(grid,),
        in_specs=[pl.BlockSpec((EDIM, _BK), lambda g: (0, g))],
        out_specs=pl.BlockSpec((_HB, 2 * EDIM), lambda g: (g, 0)),
        out_shape=jax.ShapeDtypeStruct((grid * _HB, 2 * EDIM), jnp.float32),
    )


def _gather_body(xti, t2, out, idx_v, rows_v, gsem):
    wid = lax.axis_index("s") * _NC + lax.axis_index("c")
    base = wid * _TPW
    pltpu.sync_copy(xti.at[wid], idx_v)
    gs = [
        pltpu.async_copy(
            t2.at[idx_v.at[c]],
            rows_v.at[pl.ds(c * _CHUNK, _CHUNK)],
            gsem,
        )
        for c in range(_TPW // _CHUNK)
    ]
    for g in gs:
        g.wait()
    pltpu.sync_copy(rows_v, out.at[pl.ds(base, _TPW)])


@functools.cache
def _make_gather(n_tokens, n_lines):
    mesh = plsc.VectorSubcoreMesh(core_axis_name="c", subcore_axis_name="s")
    return functools.partial(
        pl.kernel,
        out_type=jax.ShapeDtypeStruct((n_tokens, 2 * EDIM), jnp.float32),
        mesh=mesh,
        scratch_types=[
            pltpu.VMEM((8, _CHUNK), jnp.int32),
            pltpu.VMEM((_TPW, 2 * EDIM), jnp.float32),
            pltpu.SemaphoreType.DMA,
        ],
        compiler_params=pltpu.CompilerParams(use_tc_tiling_on_sc=True),
    )(_gather_body)


def _mm_body(h0, h1, h2, h3, h4, h5, h6, m_ref, w_ref, b_ref, o_ref):
    acc = b_ref[...].astype(jnp.float32)
    tm = o_ref.shape[0]
    for i, h_ref in enumerate((h0, h1, h2, h3, h4, h5, h6)):
        wide = h_ref[...]
        sel = m_ref[i].reshape(tm, 1) > 0.5
        h_i = jnp.where(sel, wide[:, EDIM:], wide[:, :EDIM]).astype(
            jnp.bfloat16)
        acc = acc + jnp.dot(h_i, w_ref[i], preferred_element_type=jnp.float32)
    o_ref[...] = acc


@functools.cache
def _make_matmul(n_tokens, tm):
    h_spec = pl.BlockSpec((tm, 2 * EDIM), lambda m: (m, 0))
    return pl.pallas_call(
        _mm_body,
        grid=(n_tokens // tm,),
        in_specs=[h_spec] * NFIELDS + [
            pl.BlockSpec((NFIELDS, tm), lambda m: (0, m)),
            pl.BlockSpec((NFIELDS, EDIM, D_MODEL), lambda m: (0, 0, 0)),
            pl.BlockSpec((1, D_MODEL), lambda m: (0, 0)),
        ],
        out_specs=pl.BlockSpec((tm, D_MODEL), lambda m: (m, 0)),
        out_shape=jax.ShapeDtypeStruct((n_tokens, D_MODEL), jnp.float32),
    )


def kernel(x, T0, T1, T2, T3, T4, T5, T6, W, b):
    bsz, seq, nf = x.shape
    n = bsz * seq
    xr = x.reshape(n, NFIELDS).astype(jnp.int32)
    line = (xr // _BK) * _HB + (xr % _HB)
    half = ((xr // _HB) & 1).astype(jnp.float32)
    xt = jnp.pad(
        line.T.reshape(NFIELDS, _NW, _TPW // _CHUNK, _CHUNK),
        ((0, 0), (0, 0), (0, 8 - _TPW // _CHUNK), (0, 0)),
    )
    m = half.T
    hs = []
    for i, T in enumerate((T0, T1, T2, T3, T4, T5, T6)):
        t2 = _make_transpose(T.shape[0])(jnp.transpose(T))
        hs.append(_make_gather(n, t2.shape[0])(xt[i], t2))
    wt = W.T.reshape(NFIELDS, EDIM, D_MODEL).astype(jnp.bfloat16)
    out = _make_matmul(n, 512)(*hs, m, wt, b.reshape(1, D_MODEL))
    return out.reshape(bsz, seq, D_MODEL)
